# Initial kernel scaffold; baseline (speedup 1.0000x reference)
import jax
import jax.numpy as jnp
from jax.experimental import pallas as pl


def kernel(x, edge_index, W0, W1, W2, b0, b1, b2):
    raise NotImplementedError("write your pallas kernel here")


# speedup vs baseline: 1.2697x; 1.2697x over previous
import functools

import jax
import jax.numpy as jnp
from jax import lax
from jax.experimental import pallas as pl
from jax.experimental.pallas import tpu as pltpu
from jax.experimental.pallas import tpu_sc as plsc

NC = 2
NS = 16
L = 16


def _mesh():
    return plsc.VectorSubcoreMesh(core_axis_name="c", subcore_axis_name="s")


@functools.cache
def _make_deg_kernel(n, e, eb):
    ept = e // (NC * NS)
    nb = ept // eb
    tr = 632
    assert NS * tr >= n and tr % 8 == 0 and (n - tr) % 8 == 0

    def body(col_hbm, out_hbm, colbuf, ones, zbuf, acc_sh):
        c = lax.axis_index("c")
        s = lax.axis_index("s")
        base = jnp.minimum(s * tr, n - tr)
        zero16 = jnp.zeros((L,), jnp.float32)
        one16 = jnp.ones((L,), jnp.float32)

        def fill(i, _):
            zbuf[i, :] = zero16
            return 0

        lax.fori_loop(0, tr, fill, 0, unroll=4)

        def fill1(i, _):
            ones[i, :] = one16
            return 0

        lax.fori_loop(0, eb, fill1, 0, unroll=4)
        pltpu.sync_copy(zbuf, acc_sh.at[pl.ds(base, tr)])
        plsc.subcore_barrier()

        pltpu.sync_copy(col_hbm.at[c, s], colbuf)

        def batch(i, _):
            pltpu.sync_copy(ones, acc_sh.at[colbuf.at[i]], add=True)
            return 0

        lax.fori_loop(0, nb, batch, 0)
        plsc.subcore_barrier()
        pltpu.sync_copy(acc_sh.at[pl.ds(base, tr)],
                        out_hbm.at[c, pl.ds(base, tr)])

    return pl.kernel(
        body,
        out_type=jax.ShapeDtypeStruct((NC, n, L), jnp.float32),
        mesh=_mesh(),
        compiler_params=pltpu.CompilerParams(needs_layout_passes=False, use_tc_tiling_on_sc=False),
        scratch_types=[
            pltpu.VMEM((nb, eb), jnp.int32),
            pltpu.VMEM((eb, L), jnp.float32),
            pltpu.VMEM((tr, L), jnp.float32),
            pltpu.VMEM_SHARED((n, L), jnp.float32),
        ],
    )


@functools.cache
def _make_dinv_kernel(n):
    def body(deg_ref, dinv_ref):
        d = deg_ref[0] + deg_ref[1]
        r = lax.rsqrt(jnp.maximum(d, 1.0))
        dinv_ref[...] = jnp.where(d > 0, r, 0.0)

    return pl.pallas_call(
        body,
        out_shape=jax.ShapeDtypeStruct((n, L), jnp.float32),
    )


@functools.cache
def _make_norm_kernel(n, e):
    ept = e // (NC * NS)
    ng = ept // L

    def body(row_hbm, col_hbm, dinv_hbm, out_hbm, dinvbuf, rbuf, cbuf, nbuf):
        c = lax.axis_index("c")
        s = lax.axis_index("s")
        w = c * NS + s
        pltpu.sync_copy(dinv_hbm, dinvbuf)
        pltpu.sync_copy(row_hbm.at[w], rbuf)
        pltpu.sync_copy(col_hbm.at[w], cbuf)

        def grp(g, _):
            sl = pl.ds(g * L, L)
            rv = rbuf[sl]
            cv = cbuf[sl]
            a = plsc.load_gather(dinvbuf, [rv])
            b = plsc.load_gather(dinvbuf, [cv])
            nbuf[sl] = -(a * b)
            return 0

        lax.fori_loop(0, ng, grp, 0, unroll=4)
        pltpu.sync_copy(nbuf, out_hbm.at[pl.ds(w * ept, ept)])

    return pl.kernel(
        body,
        out_type=jax.ShapeDtypeStruct((e,), jnp.float32),
        mesh=_mesh(),
        compiler_params=pltpu.CompilerParams(needs_layout_passes=False, use_tc_tiling_on_sc=False),
        scratch_types=[
            pltpu.VMEM((n,), jnp.float32),
            pltpu.VMEM((ept,), jnp.int32),
            pltpu.VMEM((ept,), jnp.int32),
            pltpu.VMEM((ept,), jnp.float32),
        ],
    )


@functools.cache
def _make_prop_kernel(n, e, eb, ch, first):
    ept = e // NS
    nb = ept // eb
    ng = eb // L
    tr = 632
    csz = (160, 160, 160, 152)
    assert NS * tr >= n and sum(csz) == tr
    cb = max(csz)

    def body(*refs):
        if first:
            (row_hbm, col_hbm, norm_hbm, tprev_hbm, tk_hbm,
             row_all, col_all, norm_all, rows, rows2, abuf, bbuf,
             acc_sh, sem) = refs
            tpp_hbm = None
        else:
            (row_hbm, col_hbm, norm_hbm, tprev_hbm, tpp_hbm, tk_hbm,
             row_all, col_all, norm_all, rows, rows2, abuf, bbuf,
             acc_sh, sem) = refs
        c = lax.axis_index("c")
        s = lax.axis_index("s")
        cn = (c * n).astype(jnp.int32)
        zero16 = jnp.zeros((L,), jnp.float32)

        base = jnp.minimum(s * tr, n - tr)

        def zb(i, _):
            for j in range(ch // L):
                abuf[i, pl.ds(j * L, L)] = zero16
            return 0

        lax.fori_loop(0, cb, zb, 0, unroll=4)
        coff = 0
        for sz in csz:
            pltpu.sync_copy(abuf.at[pl.ds(0, sz)],
                            acc_sh.at[pl.ds(base + coff, sz)])
            coff += sz
        plsc.subcore_barrier()

        pltpu.sync_copy(row_hbm.at[s], row_all)
        pltpu.sync_copy(col_hbm.at[s], col_all)
        pltpu.sync_copy(norm_hbm.at[s], norm_all)

        def adj(i, _):
            for g in range(ng):
                sl = pl.ds(g * L, L)
                row_all[i, sl] = row_all[i, sl] + cn
            return 0

        lax.fori_loop(0, nb, adj, 0, unroll=4)

        iota = lax.iota(jnp.int32, L)

        def batch(i, _):
            pltpu.async_copy(tprev_hbm.at[row_all.at[i]], rows, sem).wait()

            def grp(g, _):
                nv = norm_all[i, pl.ds(g * L, L)]
                ridx = iota + g * L
                for j in range(ch):
                    cidx = jnp.full((L,), j, jnp.int32)
                    v = plsc.load_gather(rows, [ridx, cidx])
                    plsc.store_scatter(rows2, [ridx, cidx], v * nv)
                return 0

            lax.fori_loop(0, ng, grp, 0)
            pltpu.sync_copy(rows2, acc_sh.at[col_all.at[i]], add=True)
            return 0

        lax.fori_loop(0, nb, batch, 0)
        plsc.subcore_barrier()

        coff = 0
        for sz in csz:
            off = base + coff
            pltpu.sync_copy(acc_sh.at[pl.ds(off, sz)], abuf.at[pl.ds(0, sz)])
            if not first:
                pltpu.sync_copy(tpp_hbm.at[pl.ds(c * n + off, sz)],
                                bbuf.at[pl.ds(0, sz)])

                def cmb(i, _):
                    for j in range(ch // L):
                        sl = pl.ds(j * L, L)
                        abuf[i, sl] = 2.0 * abuf[i, sl] - bbuf[i, sl]
                    return 0

                lax.fori_loop(0, sz, cmb, 0, unroll=4)
            pltpu.sync_copy(abuf.at[pl.ds(0, sz)],
                            tk_hbm.at[pl.ds(c * n + off, sz)])
            coff += sz

    return pl.kernel(
        body,
        out_type=jax.ShapeDtypeStruct((NC * n, ch), jnp.float32),
        mesh=_mesh(),
        compiler_params=pltpu.CompilerParams(needs_layout_passes=False, use_tc_tiling_on_sc=False),
        scratch_types=[
            pltpu.VMEM((nb, eb), jnp.int32),
            pltpu.VMEM((nb, eb), jnp.int32),
            pltpu.VMEM((nb, eb), jnp.float32),
            pltpu.VMEM((eb, ch), jnp.float32),
            pltpu.VMEM((eb, ch), jnp.float32),
            pltpu.VMEM((cb, ch), jnp.float32),
            pltpu.VMEM((cb, ch), jnp.float32),
            pltpu.VMEM_SHARED((n, ch), jnp.float32),
            pltpu.SemaphoreType.DMA,
        ],
    )


@functools.cache
def _make_matmul_kernel(n, ch, out_c, nk, rb):
    ngrid = n // rb

    def body(*refs):
        t_refs = refs[:nk]
        w_ref, b_ref, o_ref = refs[nk:]
        acc = jnp.broadcast_to(b_ref[...], (rb, out_c))
        for k in range(nk):
            for c in range(NC):
                acc = acc + jnp.dot(
                    t_refs[k][c], w_ref[k * NC + c],
                    preferred_element_type=jnp.float32,
                    precision=lax.Precision.HIGHEST)
        o_ref[...] = acc

    t_spec = pl.BlockSpec((NC, rb, ch), lambda i: (0, i, 0))
    return pl.pallas_call(
        body,
        grid=(ngrid,),
        in_specs=[t_spec] * nk + [
            pl.BlockSpec((nk * NC, ch, out_c), lambda i: (0, 0, 0)),
            pl.BlockSpec((1, out_c), lambda i: (0, 0)),
        ],
        out_specs=pl.BlockSpec((rb, out_c), lambda i: (i, 0)),
        out_shape=jax.ShapeDtypeStruct((n, out_c), jnp.float32),
    )


def kernel(x, edge_index, W0, W1, W2, b0, b1, b2):
    n, in_c = x.shape
    e = edge_index.shape[1]
    ch = in_c // NC
    eb = 80
    row = edge_index[0]
    col = edge_index[1]

    col_deg = col.reshape(NC, NS, e // (NC * NS) // eb, eb)
    deg16 = _make_deg_kernel(n, e, eb)(col_deg)
    dinv16 = _make_dinv_kernel(n)(deg16)
    dinv = dinv16[:, 0]

    row_w = row.reshape(NC * NS, e // (NC * NS))
    col_w = col.reshape(NC * NS, e // (NC * NS))
    norm = _make_norm_kernel(n, e)(row_w, col_w, dinv)

    t0 = x.reshape(n, NC, ch).transpose(1, 0, 2).reshape(NC * n, ch)
    row_t = row.reshape(NS, e // NS // eb, eb)
    col_t = col.reshape(NS, e // NS // eb, eb)
    norm_t = norm.reshape(NS, e // NS // eb, eb)

    prop_first = _make_prop_kernel(n, e, eb, ch, True)
    prop_gen = _make_prop_kernel(n, e, eb, ch, False)

    kmax = max(W0.shape[0], W1.shape[0], W2.shape[0])
    ts = [t0, prop_first(row_t, col_t, norm_t, t0)]
    for _ in range(2, kmax):
        ts.append(prop_gen(row_t, col_t, norm_t, ts[-1], ts[-2]))

    out_c = W0.shape[2] + W1.shape[2] + W2.shape[2]
    wblocks = []
    for k in range(kmax):
        for c in range(NC):
            cols = []
            for W in (W0, W1, W2):
                if k < W.shape[0]:
                    cols.append(W[k, c * ch:(c + 1) * ch, :])
                else:
                    cols.append(jnp.zeros((ch, W.shape[2]), jnp.float32))
            wblocks.append(jnp.concatenate(cols, axis=1))
    wbig = jnp.stack(wblocks)
    bias = jnp.concatenate([b0, b1, b2])[None, :]

    t_in = [t.reshape(NC, n, ch) for t in ts]
    mm = _make_matmul_kernel(n, ch, out_c, kmax, 1000)
    return mm(*t_in, wbig, bias)


# speedup vs baseline: 5.1542x; 4.0595x over previous
import functools

import jax
import jax.numpy as jnp
from jax import lax
from jax.experimental import pallas as pl
from jax.experimental.pallas import tpu as pltpu
from jax.experimental.pallas import tpu_sc as plsc

NC = 2
NS = 16
L = 16
R = 4


def _mesh():
    return plsc.VectorSubcoreMesh(core_axis_name="c", subcore_axis_name="s")


def _sc_params():
    return pltpu.CompilerParams(needs_layout_passes=False,
                                use_tc_tiling_on_sc=False)


@functools.cache
def _make_deg_kernel(n, e, eb):
    ept = e // (NC * NS)
    nb = ept // eb
    tr = 632
    assert NS * tr >= n and tr % 8 == 0 and (n - tr) % 8 == 0

    def body(col_hbm, out_hbm, colbuf, ones, zbuf, acc_sh):
        c = lax.axis_index("c")
        s = lax.axis_index("s")
        base = jnp.minimum(s * tr, n - tr)
        zero16 = jnp.zeros((L,), jnp.float32)
        one16 = jnp.ones((L,), jnp.float32)

        def fill(i, _):
            zbuf[i, :] = zero16
            return 0

        lax.fori_loop(0, tr, fill, 0, unroll=4)

        def fill1(i, _):
            ones[i, :] = one16
            return 0

        lax.fori_loop(0, eb, fill1, 0, unroll=4)
        pltpu.sync_copy(zbuf, acc_sh.at[pl.ds(base, tr)])
        plsc.subcore_barrier()

        pltpu.sync_copy(col_hbm.at[c, s], colbuf)

        def batch(i, _):
            pltpu.sync_copy(ones, acc_sh.at[colbuf.at[i]], add=True)
            return 0

        lax.fori_loop(0, nb, batch, 0)
        plsc.subcore_barrier()
        pltpu.sync_copy(acc_sh.at[pl.ds(base, tr)],
                        out_hbm.at[c, pl.ds(base, tr)])

    return pl.kernel(
        body,
        out_type=jax.ShapeDtypeStruct((NC, n, L), jnp.float32),
        mesh=_mesh(),
        compiler_params=_sc_params(),
        scratch_types=[
            pltpu.VMEM((nb, eb), jnp.int32),
            pltpu.VMEM((eb, L), jnp.float32),
            pltpu.VMEM((tr, L), jnp.float32),
            pltpu.VMEM_SHARED((n, L), jnp.float32),
        ],
    )


@functools.cache
def _make_dinv_kernel(n, in_c):
    def body(deg_ref, x_ref, dinv_ref, u0_ref):
        d = deg_ref[0] + deg_ref[1]
        r = lax.rsqrt(jnp.maximum(d, 1.0))
        dinv = jnp.where(d > 0, r, 0.0)
        dinv_ref[...] = dinv
        u0_ref[...] = x_ref[...] * dinv[:, :1]

    return pl.pallas_call(
        body,
        out_shape=(
            jax.ShapeDtypeStruct((n, L), jnp.float32),
            jax.ShapeDtypeStruct((n, in_c), jnp.float32),
        ),
    )


@functools.cache
def _make_prop_kernel(n, ep, eb, ch, first, last):
    ept = ep // NS
    nb = ept // eb
    ng = eb // L
    tr, cb = 640, 80
    ncb = tr // cb
    assert NS * tr >= n and nb % R == 0

    def body(*refs):
        it = iter(refs)
        row_hbm = next(it)
        col_hbm = next(it)
        u_hbm = next(it)
        tpp_hbm = None if first else next(it)
        dinv_hbm = next(it)
        tk_hbm = next(it)
        uk_hbm = None if last else next(it)
        row_all, col_all = next(it), next(it)
        rows = [next(it) for _ in range(R)]
        gsem = [next(it) for _ in range(R)]
        abuf, bbuf, dinvbuf, acc_sh = next(it), next(it), next(it), next(it)

        c = lax.axis_index("c")
        s = lax.axis_index("s")
        cn = (c * n).astype(jnp.int32)
        zero16 = jnp.zeros((L,), jnp.float32)
        base = jnp.minimum(s * tr, n - tr)

        def zb(i, _):
            for j in range(ch // L):
                abuf[i, pl.ds(j * L, L)] = zero16
            return 0

        lax.fori_loop(0, cb, zb, 0, unroll=4)
        for k2 in range(ncb):
            pltpu.sync_copy(abuf, acc_sh.at[pl.ds(base + k2 * cb, cb)])
        plsc.subcore_barrier()

        pltpu.sync_copy(row_hbm.at[s], row_all)
        pltpu.sync_copy(col_hbm.at[s], col_all)

        def adj(i, _):
            for g in range(ng):
                sl = pl.ds(g * L, L)
                row_all[i, sl] = row_all[i, sl] + cn
            return 0

        lax.fori_loop(0, nb, adj, 0, unroll=4)

        for p in range(R):
            pltpu.async_copy(u_hbm.at[row_all.at[p]], rows[p], gsem[p])

        def ring(m, _):
            for p in range(R):
                i = m * R + p
                pltpu.make_async_copy(
                    u_hbm.at[row_all.at[i]], rows[p], gsem[p]).wait()
                pltpu.sync_copy(rows[p], acc_sh.at[col_all.at[i]], add=True)
                nxt = i + R

                @pl.when(nxt < nb)
                def _():
                    pltpu.async_copy(
                        u_hbm.at[row_all.at[nxt]], rows[p], gsem[p])

            return 0

        lax.fori_loop(0, nb // R, ring, 0)
        plsc.subcore_barrier()

        pltpu.sync_copy(dinv_hbm.at[pl.ds(base, tr)], dinvbuf)
        iota = lax.iota(jnp.int32, L)
        for k2 in range(ncb):
            off = base + k2 * cb
            pltpu.sync_copy(acc_sh.at[pl.ds(off, cb)], abuf)
            if not first:
                pltpu.sync_copy(tpp_hbm.at[pl.ds(c * n + off, cb)], bbuf)

            def cmb(g, _):
                ridx = iota + g * L
                dv = dinvbuf[pl.ds(k2 * cb + g * L, L)]
                for j in range(ch):
                    cidx = jnp.full((L,), j, jnp.int32)
                    a = plsc.load_gather(abuf, [ridx, cidx])
                    if first:
                        t = -(dv * a)
                    else:
                        b = plsc.load_gather(bbuf, [ridx, cidx])
                        t = (-2.0) * (dv * a) - b
                    plsc.store_scatter(abuf, [ridx, cidx], t)
                    if not last:
                        plsc.store_scatter(bbuf, [ridx, cidx], dv * t)
                return 0

            lax.fori_loop(0, cb // L, cmb, 0)
            pltpu.sync_copy(abuf, tk_hbm.at[pl.ds(c * n + off, cb)])
            if not last:
                pltpu.sync_copy(bbuf, uk_hbm.at[pl.ds(c * n + off, cb)])

    n_out = 1 if last else 2
    out_type = [jax.ShapeDtypeStruct((NC * n, ch), jnp.float32)] * n_out
    return pl.kernel(
        body,
        out_type=out_type if n_out > 1 else out_type[0],
        mesh=_mesh(),
        compiler_params=_sc_params(),
        scratch_types=[
            pltpu.VMEM((nb, eb), jnp.int32),
            pltpu.VMEM((nb, eb), jnp.int32),
        ] + [pltpu.VMEM((eb, ch), jnp.float32) for _ in range(R)]
          + [pltpu.SemaphoreType.DMA for _ in range(R)] + [
            pltpu.VMEM((80, ch), jnp.float32),
            pltpu.VMEM((80, ch), jnp.float32),
            pltpu.VMEM((640,), jnp.float32),
            pltpu.VMEM_SHARED((n + 8, ch), jnp.float32),
        ],
    )


@functools.cache
def _make_matmul_kernel(n, ch, out_c, nk, rb):
    ngrid = n // rb

    def body(*refs):
        t_refs = refs[:nk]
        w_ref, b_ref, o_ref = refs[nk:]
        acc = jnp.broadcast_to(b_ref[...], (rb, out_c))
        for k in range(nk):
            for c in range(NC):
                acc = acc + jnp.dot(
                    t_refs[k][c], w_ref[k * NC + c],
                    preferred_element_type=jnp.float32,
                    precision=lax.Precision.HIGHEST)
        o_ref[...] = acc

    t_spec = pl.BlockSpec((NC, rb, ch), lambda i: (0, i, 0))
    return pl.pallas_call(
        body,
        grid=(ngrid,),
        in_specs=[t_spec] * nk + [
            pl.BlockSpec((nk * NC, ch, out_c), lambda i: (0, 0, 0)),
            pl.BlockSpec((1, out_c), lambda i: (0, 0)),
        ],
        out_specs=pl.BlockSpec((rb, out_c), lambda i: (i, 0)),
        out_shape=jax.ShapeDtypeStruct((n, out_c), jnp.float32),
    )


def kernel(x, edge_index, W0, W1, W2, b0, b1, b2):
    n, in_c = x.shape
    e = edge_index.shape[1]
    ch = in_c // NC
    row = edge_index[0]
    col = edge_index[1]

    eb_deg = 80
    col_deg = col.reshape(NC, NS, e // (NC * NS) // eb_deg, eb_deg)
    deg16 = _make_deg_kernel(n, e, eb_deg)(col_deg)
    dinv16, u0 = _make_dinv_kernel(n, in_c)(deg16, x)
    dinv = dinv16[:, 0]

    eb = 128
    quant = NS * eb * R
    ep = ((e + quant - 1) // quant) * quant
    row_p = jnp.concatenate([row, jnp.zeros((ep - e,), jnp.int32)])
    col_p = jnp.concatenate([col, jnp.full((ep - e,), n, jnp.int32)])
    row_t = row_p.reshape(NS, ep // NS // eb, eb)
    col_t = col_p.reshape(NS, ep // NS // eb, eb)

    t0 = x.reshape(n, NC, ch).transpose(1, 0, 2).reshape(NC * n, ch)
    u0 = u0.reshape(n, NC, ch).transpose(1, 0, 2).reshape(NC * n, ch)

    kmax = max(W0.shape[0], W1.shape[0], W2.shape[0])
    ts = [t0]
    us = [u0]
    for k in range(1, kmax):
        first = k == 1
        last = k == kmax - 1
        prop = _make_prop_kernel(n, ep, eb, ch, first, last)
        args = [row_t, col_t, us[-1]]
        if not first:
            args.append(ts[-2])
        args.append(dinv)
        res = prop(*args)
        if last:
            ts.append(res)
        else:
            tk, uk = res
            ts.append(tk)
            us.append(uk)

    out_c = W0.shape[2] + W1.shape[2] + W2.shape[2]
    wblocks = []
    for k in range(kmax):
        for c in range(NC):
            cols = []
            for W in (W0, W1, W2):
                if k < W.shape[0]:
                    cols.append(W[k, c * ch:(c + 1) * ch, :])
                else:
                    cols.append(jnp.zeros((ch, W.shape[2]), jnp.float32))
            wblocks.append(jnp.concatenate(cols, axis=1))
    wbig = jnp.stack(wblocks)
    bias = jnp.concatenate([b0, b1, b2])[None, :]

    t_in = [t.reshape(NC, n, ch) for t in ts]
    mm = _make_matmul_kernel(n, ch, out_c, kmax, 1000)
    return mm(*t_in, wbig, bias)
